# baseline (speedup 1.0000x reference)
import functools

import jax
import jax.numpy as jnp
from jax import lax
from jax.experimental import pallas as pl
from jax.experimental.pallas import tpu as pltpu
from jax.experimental.pallas import tpu_sc as plsc

NC, NS, LANES = 2, 16, 16
NW = NC * NS
NPTS = 1_000_000
NPAD = 1 << 20
NHP = NPAD // 2
M = 1 << 20
CHUNK = 8192
NCHH = NHP // CHUNK
NCH = NPAD // CHUNK
TSLICE = M // NS
ZCH = 8192
_FR = NPAD // 128

_mesh = plsc.VectorSubcoreMesh(core_axis_name="c", subcore_axis_name="s")

_K0R = 512


def _codes_body(b_ref, x_ref, y_ref, z_ref, out_ref):
    bi = b_ref[...].astype(jnp.int32)
    xi = (x_ref[...] * 0.5).astype(jnp.int32)
    yi = (y_ref[...] * 0.5).astype(jnp.int32)
    zi = (z_ref[...] * 0.5).astype(jnp.int32)
    out_ref[...] = ((bi * 50 + xi) * 50 + yi) * 50 + zi


def _make_codes_tc(half):
    base = half * (_FR // 2 // _K0R)

    def spec(k):
        return pl.BlockSpec(
            (_K0R, 128), lambda i, k=k: (k * (_FR // _K0R) + base + i, 0))
    return pl.pallas_call(
        _codes_body,
        grid=(_FR // 2 // _K0R,),
        in_specs=[spec(0), spec(1), spec(2), spec(3)],
        out_specs=pl.BlockSpec((_K0R, 128), lambda i: (i, 0)),
        out_shape=jax.ShapeDtypeStruct((NHP // 128, 128), jnp.int32),
    )


_codes_a = _make_codes_tc(0)
_codes_b = _make_codes_tc(1)


def _scatter_body(codes_hbm, counts_out, cbuf, zbuf, ones, table):
    c = lax.axis_index("c")
    s = lax.axis_index("s")
    wid = s * NC + c

    def fill0(i, _):
        zbuf[pl.ds(i * LANES, LANES)] = jnp.zeros((LANES,), jnp.int32)
        return 0
    lax.fori_loop(0, ZCH // LANES, fill0, 0)

    def fill1(i, _):
        ones[pl.ds(i * LANES, LANES)] = jnp.ones((LANES,), jnp.int32)
        return 0
    lax.fori_loop(0, CHUNK // LANES, fill1, 0)

    def zstep(j, _):
        pltpu.sync_copy(zbuf, table.at[pl.ds(s * TSLICE + j * ZCH, ZCH)])
        return 0
    lax.fori_loop(0, TSLICE // ZCH, zstep, 0)
    plsc.subcore_barrier()

    def step(j, _):
        off = (j * NW + wid) * CHUNK
        pltpu.sync_copy(codes_hbm.at[pl.ds(off, CHUNK)], cbuf)
        pltpu.sync_copy(ones, table.at[cbuf], add=True)
        return 0
    lax.fori_loop(0, NCHH // NW, step, 0)

    plsc.subcore_barrier()
    pltpu.sync_copy(table.at[pl.ds(s * TSLICE, TSLICE)],
                    counts_out.at[pl.ds(c * M + s * TSLICE, TSLICE)])


_scatter = functools.partial(
    pl.kernel,
    out_type=jax.ShapeDtypeStruct((NC * M,), jnp.int32),
    mesh=_mesh,
    scratch_types=[
        pltpu.VMEM((CHUNK,), jnp.int32),
        pltpu.VMEM((ZCH,), jnp.int32),
        pltpu.VMEM((CHUNK,), jnp.int32),
        pltpu.MemorySpace.VMEM_SHARED((M,), jnp.int32),
    ],
)(_scatter_body)

_ROWS, _COLS = 512, 128
_HBLK = M // (_ROWS * _COLS)


def _scan_body(a0_ref, a1_ref, b0_ref, b1_ref, out_ref, carry):
    @pl.when(pl.program_id(0) == 0)
    def _():
        carry[0] = 0

    tot = a0_ref[...] + a1_ref[...] + b0_ref[...] + b1_ref[...]
    xi = (tot > 0).astype(jnp.int32)
    rc = lax.broadcasted_iota(jnp.int32, (_COLS, _COLS), 0)
    cc = lax.broadcasted_iota(jnp.int32, (_COLS, _COLS), 1)
    tri = (rc <= cc).astype(jnp.float32)
    row_incl = jnp.dot(xi.astype(jnp.float32), tri,
                       preferred_element_type=jnp.float32).astype(jnp.int32)
    s = row_incl[:, _COLS - 1:_COLS]
    pre = jnp.concatenate(
        [jnp.zeros((1, 1), jnp.int32), s[:-1]], axis=0)
    k = 1
    while k < _ROWS:
        pre = pre + jnp.concatenate(
            [jnp.zeros((k, 1), jnp.int32), pre[:-k]], axis=0)
        k *= 2
    out_ref[...] = row_incl - xi + pre + carry[0]
    carry[0] = carry[0] + jnp.sum(xi)


_scan = pl.pallas_call(
    _scan_body,
    grid=(_HBLK,),
    in_specs=[pl.BlockSpec((_ROWS, _COLS), lambda i: (i, 0)),
              pl.BlockSpec((_ROWS, _COLS), lambda i: (i + _HBLK, 0)),
              pl.BlockSpec((_ROWS, _COLS), lambda i: (i, 0)),
              pl.BlockSpec((_ROWS, _COLS), lambda i: (i + _HBLK, 0))],
    out_specs=pl.BlockSpec((_ROWS, _COLS), lambda i: (i, 0)),
    out_shape=jax.ShapeDtypeStruct((M // _COLS, _COLS), jnp.int32),
    scratch_shapes=[pltpu.SMEM((1,), jnp.int32)],
)


@functools.partial(
    pl.kernel,
    out_type=jax.ShapeDtypeStruct((NPAD,), jnp.int32),
    mesh=_mesh,
    scratch_types=[
        pltpu.VMEM((CHUNK,), jnp.int32),
        pltpu.VMEM((CHUNK,), jnp.int32),
        pltpu.MemorySpace.VMEM_SHARED((M,), jnp.int32),
    ],
)
def _gather_kernel(codes_a_hbm, codes_b_hbm, ranks_hbm, out_hbm,
                   cbuf, gbuf, shr):
    c = lax.axis_index("c")
    s = lax.axis_index("s")
    wid = s * NC + c
    pltpu.sync_copy(ranks_hbm.at[pl.ds(s * TSLICE, TSLICE)],
                    shr.at[pl.ds(s * TSLICE, TSLICE)])
    plsc.subcore_barrier()

    def step(j, _):
        ch = j * NW + wid
        off = ch * CHUNK

        @pl.when(ch < NCHH)
        def _():
            pltpu.sync_copy(codes_a_hbm.at[pl.ds(off, CHUNK)], cbuf)

        @pl.when(ch >= NCHH)
        def _():
            pltpu.sync_copy(codes_b_hbm.at[pl.ds(off - NHP, CHUNK)], cbuf)

        pltpu.sync_copy(shr.at[cbuf], gbuf)
        pltpu.sync_copy(gbuf, out_hbm.at[pl.ds(off, CHUNK)])
        return 0
    lax.fori_loop(0, NCH // NW, step, 0)


def kernel(point_bxyz):
    flat = jnp.pad(point_bxyz.T, ((0, 0), (0, NPAD - NPTS)),
                   constant_values=8.0).reshape(4 * _FR, 128)
    codes_a = _codes_a(flat, flat, flat, flat).reshape(NHP)
    counts_a = _scatter(codes_a)
    codes_b = _codes_b(flat, flat, flat, flat).reshape(NHP)
    counts_b = _scatter(codes_b)
    ca = counts_a.reshape(2 * M // _COLS, _COLS)
    cb = counts_b.reshape(2 * M // _COLS, _COLS)
    ranks = _scan(ca, ca, cb, cb)
    return _gather_kernel(codes_a, codes_b, ranks.reshape(M))[:NPTS]


# speedup vs baseline: 1.4433x; 1.4433x over previous
import functools

import jax
import jax.numpy as jnp
from jax import lax
from jax.experimental import pallas as pl
from jax.experimental.pallas import tpu as pltpu
from jax.experimental.pallas import tpu_sc as plsc

NC, NS, LANES = 2, 16, 16
NW = NC * NS
NPTS = 1_000_000
NPAD = 1 << 20
NHP = NPAD // 2
M = 1 << 20
CHUNK = 8192
NCHH = NHP // CHUNK
NCH = NPAD // CHUNK
LASTCH = NPTS // CHUNK
LASTN = NPTS - LASTCH * CHUNK
TSLICE = M // NS
ZCH = 8192

_mesh = plsc.VectorSubcoreMesh(core_axis_name="c", subcore_axis_name="s")

_K0B = 131072


def _make_codes_tc(half):
    nb = NPAD // _K0B

    def body(b_ref, x_ref, y_ref, z_ref, out_ref):
        bi = b_ref[...].astype(jnp.int32)
        xi = (x_ref[...] * 0.5).astype(jnp.int32)
        yi = (y_ref[...] * 0.5).astype(jnp.int32)
        zi = (z_ref[...] * 0.5).astype(jnp.int32)
        code = ((bi * 50 + xi) * 50 + yi) * 50 + zi
        p = ((half * (nb // 2) + pl.program_id(0)) * _K0B
             + lax.broadcasted_iota(jnp.int32, (_K0B,), 0))
        out_ref[...] = jnp.where(p < NPTS, code, p)

    def spec(k):
        return pl.BlockSpec(
            (_K0B,), lambda i, k=k: (k * nb + half * (nb // 2) + i,))
    return pl.pallas_call(
        body,
        grid=(nb // 2,),
        in_specs=[spec(0), spec(1), spec(2), spec(3)],
        out_specs=pl.BlockSpec((_K0B,), lambda i: (i,)),
        out_shape=jax.ShapeDtypeStruct((NHP,), jnp.int32),
    )


_codes_a = _make_codes_tc(0)
_codes_b = _make_codes_tc(1)


def _scatter_body(nreal, codes_hbm, counts_out, cbuf, zbuf, ones, table):
    c = lax.axis_index("c")
    s = lax.axis_index("s")
    wid = s * NC + c

    def fill0(i, _):
        zbuf[pl.ds(i * LANES, LANES)] = jnp.zeros((LANES,), jnp.int32)
        return 0
    lax.fori_loop(0, ZCH // LANES, fill0, 0)

    def fill1(i, _):
        ones[pl.ds(i * LANES, LANES)] = jnp.ones((LANES,), jnp.int32)
        return 0
    lax.fori_loop(0, CHUNK // LANES, fill1, 0)

    def zstep(j, _):
        pltpu.sync_copy(zbuf, table.at[pl.ds(s * TSLICE + j * ZCH, ZCH)])
        return 0
    lax.fori_loop(0, TSLICE // ZCH, zstep, 0)
    plsc.subcore_barrier()

    def step(j, _):
        off = (j * NW + wid) * CHUNK

        @pl.when(off < nreal)
        def _():
            pltpu.sync_copy(codes_hbm.at[pl.ds(off, CHUNK)], cbuf)
            pltpu.sync_copy(ones, table.at[cbuf], add=True)
        return 0
    lax.fori_loop(0, NCHH // NW, step, 0)

    plsc.subcore_barrier()
    pltpu.sync_copy(table.at[pl.ds(s * TSLICE, TSLICE)],
                    counts_out.at[pl.ds(c * M + s * TSLICE, TSLICE)])


def _make_scatter(nreal):
    return functools.partial(
        pl.kernel,
        out_type=jax.ShapeDtypeStruct((NC * M,), jnp.int32),
        mesh=_mesh,
        scratch_types=[
            pltpu.VMEM((CHUNK,), jnp.int32),
            pltpu.VMEM((ZCH,), jnp.int32),
            pltpu.VMEM((CHUNK,), jnp.int32),
            pltpu.MemorySpace.VMEM_SHARED((M,), jnp.int32),
        ],
    )(functools.partial(_scatter_body, nreal))


_scatter_a = _make_scatter(NHP)
_scatter_b = _make_scatter(NPTS - NHP)

_ROWS, _COLS = 512, 128
_HBLK = M // (_ROWS * _COLS)


def _scan_body(a0_ref, a1_ref, b0_ref, b1_ref, out_ref, carry):
    @pl.when(pl.program_id(0) == 0)
    def _():
        carry[0] = 0

    tot = a0_ref[...] + a1_ref[...] + b0_ref[...] + b1_ref[...]
    xi = (tot > 0).astype(jnp.int32)
    rc = lax.broadcasted_iota(jnp.int32, (_COLS, _COLS), 0)
    cc = lax.broadcasted_iota(jnp.int32, (_COLS, _COLS), 1)
    tri = (rc <= cc).astype(jnp.float32)
    row_incl = jnp.dot(xi.astype(jnp.float32), tri,
                       preferred_element_type=jnp.float32).astype(jnp.int32)
    s = row_incl[:, _COLS - 1:_COLS]
    pre = jnp.concatenate(
        [jnp.zeros((1, 1), jnp.int32), s[:-1]], axis=0)
    k = 1
    while k < _ROWS:
        pre = pre + jnp.concatenate(
            [jnp.zeros((k, 1), jnp.int32), pre[:-k]], axis=0)
        k *= 2
    out_ref[...] = row_incl - xi + pre + carry[0]
    carry[0] = carry[0] + jnp.sum(xi)


_scan = pl.pallas_call(
    _scan_body,
    grid=(_HBLK,),
    in_specs=[pl.BlockSpec((_ROWS, _COLS), lambda i: (i, 0)),
              pl.BlockSpec((_ROWS, _COLS), lambda i: (i + _HBLK, 0)),
              pl.BlockSpec((_ROWS, _COLS), lambda i: (i, 0)),
              pl.BlockSpec((_ROWS, _COLS), lambda i: (i + _HBLK, 0))],
    out_specs=pl.BlockSpec((_ROWS, _COLS), lambda i: (i, 0)),
    out_shape=jax.ShapeDtypeStruct((M // _COLS, _COLS), jnp.int32),
    scratch_shapes=[pltpu.SMEM((1,), jnp.int32)],
)


@functools.partial(
    pl.kernel,
    out_type=jax.ShapeDtypeStruct((NPTS,), jnp.int32),
    mesh=_mesh,
    scratch_types=[
        pltpu.VMEM((CHUNK,), jnp.int32),
        pltpu.VMEM((CHUNK,), jnp.int32),
        pltpu.MemorySpace.VMEM_SHARED((M,), jnp.int32),
    ],
)
def _gather_kernel(codes_a_hbm, codes_b_hbm, ranks_hbm, out_hbm,
                   cbuf, gbuf, shr):
    c = lax.axis_index("c")
    s = lax.axis_index("s")
    wid = s * NC + c
    pltpu.sync_copy(ranks_hbm.at[pl.ds(s * TSLICE, TSLICE)],
                    shr.at[pl.ds(s * TSLICE, TSLICE)])
    plsc.subcore_barrier()

    def step(j, _):
        ch = j * NW + wid
        off = ch * CHUNK

        @pl.when(ch <= LASTCH)
        def _():
            @pl.when(ch < NCHH)
            def _():
                pltpu.sync_copy(codes_a_hbm.at[pl.ds(off, CHUNK)], cbuf)

            @pl.when(ch >= NCHH)
            def _():
                pltpu.sync_copy(codes_b_hbm.at[pl.ds(off - NHP, CHUNK)], cbuf)

            pltpu.sync_copy(shr.at[cbuf], gbuf)

            @pl.when(ch < LASTCH)
            def _():
                pltpu.sync_copy(gbuf, out_hbm.at[pl.ds(off, CHUNK)])

            @pl.when(ch == LASTCH)
            def _():
                pltpu.sync_copy(gbuf.at[pl.ds(0, LASTN)],
                                out_hbm.at[pl.ds(off, LASTN)])
        return 0
    lax.fori_loop(0, NCH // NW, step, 0)


def kernel(point_bxyz):
    flat = jnp.pad(point_bxyz.T, ((0, 0), (0, NPAD - NPTS)),
                   constant_values=8.0).reshape(4 * NPAD)
    codes_a = _codes_a(flat, flat, flat, flat)
    counts_a = _scatter_a(codes_a)
    codes_b = _codes_b(flat, flat, flat, flat)
    counts_b = _scatter_b(codes_b)
    ca = counts_a.reshape(2 * M // _COLS, _COLS)
    cb = counts_b.reshape(2 * M // _COLS, _COLS)
    ranks = _scan(ca, ca, cb, cb)
    return _gather_kernel(codes_a, codes_b, ranks.reshape(M))


# speedup vs baseline: 1.5601x; 1.0809x over previous
import functools

import jax
import jax.numpy as jnp
from jax import lax
from jax.experimental import pallas as pl
from jax.experimental.pallas import tpu as pltpu
from jax.experimental.pallas import tpu_sc as plsc

NC, NS, LANES = 2, 16, 16
NW = NC * NS
NPTS = 1_000_000
NA = 512_000
NB = NPTS - NA
M = 1 << 20
CHUNK = 8000
NCH = NPTS // CHUNK
TSLICE = M // NS
ZCH = 8192

_mesh = plsc.VectorSubcoreMesh(core_axis_name="c", subcore_axis_name="s")

_K0B = 131072


def _codes_body(b_ref, x_ref, y_ref, z_ref, out_ref):
    bi = b_ref[...].astype(jnp.int32)
    xi = (x_ref[...] * 0.5).astype(jnp.int32)
    yi = (y_ref[...] * 0.5).astype(jnp.int32)
    zi = (z_ref[...] * 0.5).astype(jnp.int32)
    out_ref[...] = ((bi * 50 + xi) * 50 + yi) * 50 + zi


def _make_codes_tc(n):
    return pl.pallas_call(
        _codes_body,
        grid=(pl.cdiv(n, _K0B),),
        in_specs=[pl.BlockSpec((_K0B,), lambda i: (i,))] * 4,
        out_specs=pl.BlockSpec((_K0B,), lambda i: (i,)),
        out_shape=jax.ShapeDtypeStruct((n,), jnp.int32),
    )


_codes_a = _make_codes_tc(NA)
_codes_b = _make_codes_tc(NB)


def _scatter_body(nchunks, codes_hbm, counts_out,
                  cbuf0, cbuf1, zbuf, ones, table, sem_z, sem_c0, sem_c1):
    c = lax.axis_index("c")
    s = lax.axis_index("s")
    wid = s * NC + c
    has1 = (NW + wid) < nchunks

    def fill0(i, _):
        zbuf[pl.ds(i * LANES, LANES)] = jnp.zeros((LANES,), jnp.int32)
        return 0
    lax.fori_loop(0, ZCH // LANES, fill0, 0)

    def fill1(i, _):
        ones[pl.ds(i * LANES, LANES)] = jnp.ones((LANES,), jnp.int32)
        return 0
    lax.fori_loop(0, CHUNK // LANES, fill1, 0)

    pltpu.async_copy(codes_hbm.at[pl.ds(wid * CHUNK, CHUNK)], cbuf0, sem_c0)

    @pl.when(has1)
    def _():
        pltpu.async_copy(codes_hbm.at[pl.ds((NW + wid) * CHUNK, CHUNK)],
                         cbuf1, sem_c1)

    def zfire(j, _):
        pltpu.async_copy(zbuf, table.at[pl.ds(s * TSLICE + j * ZCH, ZCH)],
                         sem_z)
        return 0
    lax.fori_loop(0, TSLICE // ZCH, zfire, 0)

    def zdrain(j, _):
        pltpu.make_async_copy(zbuf, table.at[pl.ds(s * TSLICE, ZCH)],
                              sem_z).wait()
        return 0
    lax.fori_loop(0, TSLICE // ZCH, zdrain, 0)
    plsc.subcore_barrier()

    pltpu.make_async_copy(codes_hbm.at[pl.ds(wid * CHUNK, CHUNK)],
                          cbuf0, sem_c0).wait()
    pltpu.sync_copy(ones, table.at[cbuf0], add=True)

    @pl.when(has1)
    def _():
        pltpu.make_async_copy(codes_hbm.at[pl.ds((NW + wid) * CHUNK, CHUNK)],
                              cbuf1, sem_c1).wait()
        pltpu.sync_copy(ones, table.at[cbuf1], add=True)

    plsc.subcore_barrier()
    pltpu.sync_copy(table.at[pl.ds(s * TSLICE, TSLICE)],
                    counts_out.at[pl.ds(c * M + s * TSLICE, TSLICE)])


def _make_scatter(n):
    return functools.partial(
        pl.kernel,
        out_type=jax.ShapeDtypeStruct((NC * M,), jnp.int32),
        mesh=_mesh,
        scratch_types=[
            pltpu.VMEM((CHUNK,), jnp.int32),
            pltpu.VMEM((CHUNK,), jnp.int32),
            pltpu.VMEM((ZCH,), jnp.int32),
            pltpu.VMEM((CHUNK,), jnp.int32),
            pltpu.MemorySpace.VMEM_SHARED((M,), jnp.int32),
            pltpu.SemaphoreType.DMA,
            pltpu.SemaphoreType.DMA,
            pltpu.SemaphoreType.DMA,
        ],
    )(functools.partial(_scatter_body, n // CHUNK))


_scatter_a = _make_scatter(NA)
_scatter_b = _make_scatter(NB)

_ROWS, _COLS = 512, 128
_HBLK = M // (_ROWS * _COLS)


def _scan_body(a0_ref, a1_ref, b0_ref, b1_ref, out_ref, carry):
    @pl.when(pl.program_id(0) == 0)
    def _():
        carry[0] = 0

    tot = a0_ref[...] + a1_ref[...] + b0_ref[...] + b1_ref[...]
    xi = (tot > 0).astype(jnp.int32)
    rc = lax.broadcasted_iota(jnp.int32, (_COLS, _COLS), 0)
    cc = lax.broadcasted_iota(jnp.int32, (_COLS, _COLS), 1)
    tri = (rc <= cc).astype(jnp.float32)
    row_incl = jnp.dot(xi.astype(jnp.float32), tri,
                       preferred_element_type=jnp.float32).astype(jnp.int32)
    s = row_incl[:, _COLS - 1:_COLS]
    pre = jnp.concatenate(
        [jnp.zeros((1, 1), jnp.int32), s[:-1]], axis=0)
    k = 1
    while k < _ROWS:
        pre = pre + jnp.concatenate(
            [jnp.zeros((k, 1), jnp.int32), pre[:-k]], axis=0)
        k *= 2
    out_ref[...] = row_incl - xi + pre + carry[0]
    carry[0] = carry[0] + jnp.sum(xi)


_scan = pl.pallas_call(
    _scan_body,
    grid=(_HBLK,),
    in_specs=[pl.BlockSpec((_ROWS, _COLS), lambda i: (i, 0)),
              pl.BlockSpec((_ROWS, _COLS), lambda i: (i + _HBLK, 0)),
              pl.BlockSpec((_ROWS, _COLS), lambda i: (i, 0)),
              pl.BlockSpec((_ROWS, _COLS), lambda i: (i + _HBLK, 0))],
    out_specs=pl.BlockSpec((_ROWS, _COLS), lambda i: (i, 0)),
    out_shape=jax.ShapeDtypeStruct((M // _COLS, _COLS), jnp.int32),
    scratch_shapes=[pltpu.SMEM((1,), jnp.int32)],
)

_ACH = NA // CHUNK


@functools.partial(
    pl.kernel,
    out_type=jax.ShapeDtypeStruct((NPTS,), jnp.int32),
    mesh=_mesh,
    scratch_types=[
        pltpu.VMEM((CHUNK,), jnp.int32),
        pltpu.VMEM((CHUNK,), jnp.int32),
        pltpu.VMEM((CHUNK,), jnp.int32),
        pltpu.VMEM((CHUNK,), jnp.int32),
        pltpu.MemorySpace.VMEM_SHARED((M,), jnp.int32),
        pltpu.SemaphoreType.DMA,
        pltpu.SemaphoreType.DMA,
        pltpu.SemaphoreType.DMA,
        pltpu.SemaphoreType.DMA,
        pltpu.SemaphoreType.DMA,
    ],
)
def _gather_kernel(codes_a_hbm, codes_b_hbm, ranks_hbm, out_hbm,
                   cbuf0, cbuf1, gbuf0, gbuf1, shr,
                   sem_st, sem_c0, sem_c1, sem_o0, sem_o1):
    c = lax.axis_index("c")
    s = lax.axis_index("s")
    wid = s * NC + c
    cbuf = (cbuf0, cbuf1)
    gbuf = (gbuf0, gbuf1)
    sem_c = (sem_c0, sem_c1)
    sem_o = (sem_o0, sem_o1)

    def src(j):
        ch_off = (j * NW + wid) * CHUNK
        if j < 2:
            return codes_a_hbm, ch_off
        return codes_b_hbm, ch_off - NA

    pltpu.async_copy(ranks_hbm.at[pl.ds(s * TSLICE, TSLICE)],
                     shr.at[pl.ds(s * TSLICE, TSLICE)], sem_st)
    r0, o0 = src(0)
    pltpu.async_copy(r0.at[pl.ds(o0, CHUNK)], cbuf0, sem_c0)
    pltpu.make_async_copy(ranks_hbm.at[pl.ds(s * TSLICE, TSLICE)],
                          shr.at[pl.ds(s * TSLICE, TSLICE)], sem_st).wait()
    plsc.subcore_barrier()

    last = NCH - 3 * NW

    for j in range(4):
        b = j % 2
        live = (wid < last) if j == 3 else True
        rj, oj = src(j)
        out_off = (j * NW + wid) * CHUNK

        def _step(b=b, j=j, rj=rj, oj=oj, out_off=out_off):
            pltpu.make_async_copy(rj.at[pl.ds(oj, CHUNK)], cbuf[b],
                                  sem_c[b]).wait()
            if j + 1 < 4:
                rn, on = src(j + 1)
                nxt = (wid < last) if j + 1 == 3 else True
                if j + 1 == 3:
                    @pl.when(nxt)
                    def _():
                        pltpu.async_copy(rn.at[pl.ds(on, CHUNK)],
                                         cbuf[1 - b], sem_c[1 - b])
                else:
                    pltpu.async_copy(rn.at[pl.ds(on, CHUNK)],
                                     cbuf[1 - b], sem_c[1 - b])
            if j >= 2:
                pltpu.make_async_copy(
                    gbuf[b], out_hbm.at[pl.ds(out_off, CHUNK)],
                    sem_o[b]).wait()
            pltpu.sync_copy(shr.at[cbuf[b]], gbuf[b])
            pltpu.async_copy(gbuf[b], out_hbm.at[pl.ds(out_off, CHUNK)],
                             sem_o[b])

        if j == 3:
            @pl.when(live)
            def _():
                _step()
        else:
            _step()

    pltpu.make_async_copy(gbuf0, out_hbm.at[pl.ds(wid * CHUNK, CHUNK)],
                          sem_o0).wait()
    pltpu.make_async_copy(gbuf1, out_hbm.at[pl.ds(wid * CHUNK, CHUNK)],
                          sem_o1).wait()


def kernel(point_bxyz):
    cols_a = [point_bxyz[:NA, k] for k in range(4)]
    codes_a = _codes_a(*cols_a)
    counts_a = _scatter_a(codes_a)
    cols_b = [point_bxyz[NA:, k] for k in range(4)]
    codes_b = _codes_b(*cols_b)
    counts_b = _scatter_b(codes_b)
    ca = counts_a.reshape(2 * M // _COLS, _COLS)
    cb = counts_b.reshape(2 * M // _COLS, _COLS)
    ranks = _scan(ca, ca, cb, cb)
    return _gather_kernel(codes_a, codes_b, ranks.reshape(M))
